# baseline (speedup 1.0000x reference)
import functools

import jax
import jax.numpy as jnp
from jax import lax
from jax.experimental import pallas as pl
from jax.experimental.pallas import tpu as pltpu
from jax.experimental.pallas import tpu_sc as plsc

VEC = 16
NC, NS = 2, 16
NW = NC * NS
CH_R = 16


def _sampler_body(vocab, n_cols, n_chunks, p_hbm, t_hbm, v_hbm, out_hbm,
                  p_v0, p_v1, i_v0, i_v1, tb_v0, tb_v1, j_v0, j_v1,
                  o_v0, o_v1, t_sh, v_sh,
                  p_sem, t_sem, g_sem, o_sem0, o_sem1):
    cid = lax.axis_index("c")
    sid = lax.axis_index("s")
    wid = cid * NS + sid
    row0 = wid * (n_chunks * CH_R)
    n_full = n_cols // VEC
    tail = n_cols - n_full * VEC
    pstride = -(-n_cols // 128) * 128
    vocab_f = jnp.float32(vocab)
    n_pairs = n_chunks // 2

    pltpu.async_copy(p_hbm.at[pl.ds(row0, CH_R)], p_v0, p_sem)

    @pl.when(sid == 0)
    def _():
        pltpu.sync_copy(v_hbm, v_sh)

    @pl.when(sid == 1)
    def _():
        pltpu.sync_copy(t_hbm, t_sh)

    plsc.subcore_barrier()

    lanes = lax.iota(jnp.int32, VEC)
    hi = (lanes >= tail).astype(jnp.int32) if tail else None
    cvec = (n_full * VEC + (lanes - hi * tail)) if tail else None

    def gather_pieces(j_v, o_v, fire):
        copies = []
        for r in range(CH_R):
            for c0 in range(0, n_cols, 128):
                w = min(128, n_cols - c0)
                src = v_sh.at[j_v.at[pl.ds(r * pstride + c0, w)]]
                dst = o_v.at[r, pl.ds(c0, w)]
                if fire:
                    copies.append(pltpu.async_copy(src, dst, g_sem))
                else:
                    copies.append(pltpu.make_async_copy(src, dst, g_sem))
        return copies

    def do_chunk(g, not_first, has_prev, p_v, i_v, tb_v, j_v, o_v, o_sem,
                 j_prev, o_prev, o_sem_prev):
        pltpu.make_async_copy(p_hbm.at[pl.ds(row0, CH_R)], p_v, p_sem).wait()

        @plsc.parallel_loop(0, CH_R, 1, unroll=4)
        def _(r):
            for c in range(n_full):
                p = p_v[r, pl.ds(c * VEC, VEC)] * vocab_f
                plsc.store_scatter(i_v, [r * n_cols + c * VEC + lanes],
                                   p.astype(jnp.int32))

        if tail:
            @plsc.parallel_loop(0, CH_R // 2, 1, unroll=4)
            def _(u):
                rvec = 2 * u + hi
                p = plsc.load_gather(p_v, [rvec, cvec]) * vocab_f
                plsc.store_scatter(i_v, [rvec * n_cols + cvec],
                                   p.astype(jnp.int32))

        pltpu.async_copy(t_sh.at[i_v], tb_v, t_sem).wait()

        @plsc.parallel_loop(0, CH_R, 1, unroll=4)
        def _(r):
            for c in range(n_full):
                didx = r * n_cols + c * VEC + lanes
                p = p_v[r, pl.ds(c * VEC, VEC)] * vocab_f
                i = plsc.load_gather(i_v, [didx])
                t = plsc.load_gather(tb_v, [didx])
                frac = p - i.astype(jnp.float32)
                j = i + i + jnp.where(t < frac, 1, 0)
                plsc.store_scatter(j_v, [r * pstride + c * VEC + lanes], j)

        if tail:
            @plsc.parallel_loop(0, CH_R // 2, 1, unroll=4)
            def _(u):
                rvec = 2 * u + hi
                didx = rvec * n_cols + cvec
                p = plsc.load_gather(p_v, [rvec, cvec]) * vocab_f
                i = plsc.load_gather(i_v, [didx])
                t = plsc.load_gather(tb_v, [didx])
                frac = p - i.astype(jnp.float32)
                j = i + i + jnp.where(t < frac, 1, 0)
                plsc.store_scatter(j_v, [rvec * pstride + cvec], j)

        @pl.when(not_first)
        def _():
            pltpu.make_async_copy(o_v, out_hbm.at[pl.ds(row0, CH_R)],
                                  o_sem).wait()
        gather_pieces(j_v, o_v, fire=True)

        @pl.when(has_prev)
        def _():
            for cp in gather_pieces(j_prev, o_prev, fire=False):
                cp.wait()
            pltpu.async_copy(o_prev,
                             out_hbm.at[pl.ds(row0 + (g - 1) * CH_R, CH_R)],
                             o_sem_prev)

    def pair_body(k, carry):
        g0 = 2 * k
        pltpu.async_copy(p_hbm.at[pl.ds(row0 + (g0 + 1) * CH_R, CH_R)],
                         p_v1, p_sem)
        do_chunk(g0, k >= 1, k >= 1, p_v0, i_v0, tb_v0, j_v0, o_v0, o_sem0,
                 j_v1, o_v1, o_sem1)

        @pl.when(k < n_pairs - 1)
        def _():
            pltpu.async_copy(p_hbm.at[pl.ds(row0 + (g0 + 2) * CH_R, CH_R)],
                             p_v0, p_sem)

        do_chunk(g0 + 1, k >= 1, True, p_v1, i_v1, tb_v1, j_v1, o_v1,
                 o_sem1, j_v0, o_v0, o_sem0)
        return carry

    lax.fori_loop(0, n_pairs, pair_body, 0)

    for cp in gather_pieces(j_v1, o_v1, fire=False):
        cp.wait()
    pltpu.async_copy(o_v1, out_hbm.at[pl.ds(row0 + (n_chunks - 1) * CH_R,
                                            CH_R)], o_sem1)
    pltpu.make_async_copy(o_v0, out_hbm.at[pl.ds(row0, CH_R)], o_sem0).wait()
    pltpu.make_async_copy(o_v1, out_hbm.at[pl.ds(row0, CH_R)], o_sem1).wait()


def kernel(p_unit, threshold, values):
    batch, n_samples = p_unit.shape
    vocab = threshold.shape[0]
    assert batch % (NW * 2 * CH_R) == 0
    n_chunks = batch // (NW * CH_R)
    ch_e = CH_R * n_samples
    ch_pad = CH_R * (-(-n_samples // 128) * 128)

    mesh = plsc.VectorSubcoreMesh(core_axis_name="c", subcore_axis_name="s")
    run = functools.partial(
        pl.kernel,
        mesh=mesh,
        compiler_params=pltpu.CompilerParams(
            needs_layout_passes=False,
            disable_bounds_checks=True,
            disable_semaphore_checks=True,
        ),
        out_type=jax.ShapeDtypeStruct((batch, n_samples), jnp.int32),
        scratch_types=[
            pltpu.VMEM((CH_R, n_samples), jnp.float32),
            pltpu.VMEM((CH_R, n_samples), jnp.float32),
            pltpu.VMEM((ch_e,), jnp.int32),
            pltpu.VMEM((ch_e,), jnp.int32),
            pltpu.VMEM((ch_e,), jnp.float32),
            pltpu.VMEM((ch_e,), jnp.float32),
            pltpu.VMEM((ch_pad,), jnp.int32),
            pltpu.VMEM((ch_pad,), jnp.int32),
            pltpu.VMEM((CH_R, n_samples), jnp.int32),
            pltpu.VMEM((CH_R, n_samples), jnp.int32),
            pltpu.VMEM_SHARED((vocab,), jnp.float32),
            pltpu.VMEM_SHARED((2 * vocab,), jnp.int32),
            pltpu.SemaphoreType.DMA,
            pltpu.SemaphoreType.DMA,
            pltpu.SemaphoreType.DMA,
            pltpu.SemaphoreType.DMA,
            pltpu.SemaphoreType.DMA,
        ],
    )(functools.partial(_sampler_body, vocab, n_samples, n_chunks))

    return run(p_unit, threshold, values)


# speedup vs baseline: 1.0311x; 1.0311x over previous
import functools

import jax
import jax.numpy as jnp
from jax import lax
from jax.experimental import pallas as pl
from jax.experimental.pallas import tpu as pltpu
from jax.experimental.pallas import tpu_sc as plsc

VEC = 16
NC, NS = 2, 16
NW = NC * NS
CH_R = 16


def _sampler_body(vocab, n_cols, n_chunks, p_hbm, t_hbm, v_hbm, out_hbm,
                  p_v0, p_v1, i_v0, i_v1, tb_v0, tb_v1, j_v0, j_v1,
                  o_v0, o_v1, t_sh, v_sh,
                  p_sem, t_sem, g_sem, o_sem0, o_sem1):
    cid = lax.axis_index("c")
    sid = lax.axis_index("s")
    wid = cid * NS + sid
    row0 = wid * (n_chunks * CH_R)
    n_full = n_cols // VEC
    tail = n_cols - n_full * VEC
    pstride = -(-n_cols // 128) * 128
    vocab_f = jnp.float32(vocab)
    n_pairs = n_chunks // 2

    pltpu.async_copy(p_hbm.at[pl.ds(row0, CH_R)], p_v0, p_sem)

    @pl.when(sid == 0)
    def _():
        pltpu.sync_copy(v_hbm, v_sh)

    @pl.when(sid == 1)
    def _():
        pltpu.sync_copy(t_hbm, t_sh)

    plsc.subcore_barrier()

    lanes = lax.iota(jnp.int32, VEC)
    hi = (lanes >= tail).astype(jnp.int32) if tail else None
    cvec = (n_full * VEC + (lanes - hi * tail)) if tail else None

    def gather_pieces(j_v, o_v, fire):
        copies = []
        for r in range(CH_R):
            for c0 in range(0, n_cols, 128):
                w = min(128, n_cols - c0)
                src = v_sh.at[j_v.at[pl.ds(r * pstride + c0, w)]]
                dst = o_v.at[r, pl.ds(c0, w)]
                if fire:
                    copies.append(pltpu.async_copy(src, dst, g_sem))
                else:
                    copies.append(pltpu.make_async_copy(src, dst, g_sem))
        return copies

    def do_chunk(g, not_first, has_prev, p_v, i_v, tb_v, j_v, o_v, o_sem,
                 j_prev, o_prev, o_sem_prev):
        pltpu.make_async_copy(p_hbm.at[pl.ds(row0, CH_R)], p_v, p_sem).wait()

        @plsc.parallel_loop(0, CH_R, 1, unroll=2)
        def _(r):
            for c in range(n_full):
                p = p_v[r, pl.ds(c * VEC, VEC)] * vocab_f
                plsc.store_scatter(i_v, [r * n_cols + c * VEC + lanes],
                                   p.astype(jnp.int32))

        if tail:
            @plsc.parallel_loop(0, CH_R // 2, 1, unroll=2)
            def _(u):
                rvec = 2 * u + hi
                p = plsc.load_gather(p_v, [rvec, cvec]) * vocab_f
                plsc.store_scatter(i_v, [rvec * n_cols + cvec],
                                   p.astype(jnp.int32))

        pltpu.async_copy(t_sh.at[i_v], tb_v, t_sem).wait()

        @plsc.parallel_loop(0, CH_R, 1, unroll=2)
        def _(r):
            for c in range(n_full):
                didx = r * n_cols + c * VEC + lanes
                p = p_v[r, pl.ds(c * VEC, VEC)] * vocab_f
                i = plsc.load_gather(i_v, [didx])
                t = plsc.load_gather(tb_v, [didx])
                frac = p - i.astype(jnp.float32)
                j = i + i + jnp.where(t < frac, 1, 0)
                plsc.store_scatter(j_v, [r * pstride + c * VEC + lanes], j)

        if tail:
            @plsc.parallel_loop(0, CH_R // 2, 1, unroll=2)
            def _(u):
                rvec = 2 * u + hi
                didx = rvec * n_cols + cvec
                p = plsc.load_gather(p_v, [rvec, cvec]) * vocab_f
                i = plsc.load_gather(i_v, [didx])
                t = plsc.load_gather(tb_v, [didx])
                frac = p - i.astype(jnp.float32)
                j = i + i + jnp.where(t < frac, 1, 0)
                plsc.store_scatter(j_v, [rvec * pstride + cvec], j)

        @pl.when(not_first)
        def _():
            pltpu.make_async_copy(o_v, out_hbm.at[pl.ds(row0, CH_R)],
                                  o_sem).wait()
        gather_pieces(j_v, o_v, fire=True)

        @pl.when(has_prev)
        def _():
            for cp in gather_pieces(j_prev, o_prev, fire=False):
                cp.wait()
            pltpu.async_copy(o_prev,
                             out_hbm.at[pl.ds(row0 + (g - 1) * CH_R, CH_R)],
                             o_sem_prev)

    def pair_body(k, carry):
        g0 = 2 * k
        pltpu.async_copy(p_hbm.at[pl.ds(row0 + (g0 + 1) * CH_R, CH_R)],
                         p_v1, p_sem)
        do_chunk(g0, k >= 1, k >= 1, p_v0, i_v0, tb_v0, j_v0, o_v0, o_sem0,
                 j_v1, o_v1, o_sem1)

        @pl.when(k < n_pairs - 1)
        def _():
            pltpu.async_copy(p_hbm.at[pl.ds(row0 + (g0 + 2) * CH_R, CH_R)],
                             p_v0, p_sem)

        do_chunk(g0 + 1, k >= 1, True, p_v1, i_v1, tb_v1, j_v1, o_v1,
                 o_sem1, j_v0, o_v0, o_sem0)
        return carry

    lax.fori_loop(0, n_pairs, pair_body, 0)

    for cp in gather_pieces(j_v1, o_v1, fire=False):
        cp.wait()
    pltpu.async_copy(o_v1, out_hbm.at[pl.ds(row0 + (n_chunks - 1) * CH_R,
                                            CH_R)], o_sem1)
    pltpu.make_async_copy(o_v0, out_hbm.at[pl.ds(row0, CH_R)], o_sem0).wait()
    pltpu.make_async_copy(o_v1, out_hbm.at[pl.ds(row0, CH_R)], o_sem1).wait()


def kernel(p_unit, threshold, values):
    batch, n_samples = p_unit.shape
    vocab = threshold.shape[0]
    assert batch % (NW * 2 * CH_R) == 0
    n_chunks = batch // (NW * CH_R)
    ch_e = CH_R * n_samples
    ch_pad = CH_R * (-(-n_samples // 128) * 128)

    mesh = plsc.VectorSubcoreMesh(core_axis_name="c", subcore_axis_name="s")
    run = functools.partial(
        pl.kernel,
        mesh=mesh,
        compiler_params=pltpu.CompilerParams(
            needs_layout_passes=False,
            disable_bounds_checks=True,
            disable_semaphore_checks=True,
        ),
        out_type=jax.ShapeDtypeStruct((batch, n_samples), jnp.int32),
        scratch_types=[
            pltpu.VMEM((CH_R, n_samples), jnp.float32),
            pltpu.VMEM((CH_R, n_samples), jnp.float32),
            pltpu.VMEM((ch_e,), jnp.int32),
            pltpu.VMEM((ch_e,), jnp.int32),
            pltpu.VMEM((ch_e,), jnp.float32),
            pltpu.VMEM((ch_e,), jnp.float32),
            pltpu.VMEM((ch_pad,), jnp.int32),
            pltpu.VMEM((ch_pad,), jnp.int32),
            pltpu.VMEM((CH_R, n_samples), jnp.int32),
            pltpu.VMEM((CH_R, n_samples), jnp.int32),
            pltpu.VMEM_SHARED((vocab,), jnp.float32),
            pltpu.VMEM_SHARED((2 * vocab,), jnp.int32),
            pltpu.SemaphoreType.DMA,
            pltpu.SemaphoreType.DMA,
            pltpu.SemaphoreType.DMA,
            pltpu.SemaphoreType.DMA,
            pltpu.SemaphoreType.DMA,
        ],
    )(functools.partial(_sampler_body, vocab, n_samples, n_chunks))

    return run(p_unit, threshold, values)
